# Initial kernel scaffold; baseline (speedup 1.0000x reference)
import jax
import jax.numpy as jnp
from jax.experimental import pallas as pl


def kernel(x, edge_index, W_l1, W_r1, b1, W_l2, W_r2, b2):
    raise NotImplementedError("write your pallas kernel here")


# speedup vs baseline: 13.8510x; 13.8510x over previous
import functools

import jax
import jax.numpy as jnp
from jax import lax
from jax.experimental import pallas as pl
from jax.experimental.pallas import tpu as pltpu
from jax.experimental.pallas import tpu_sc as plsc

N = 10000
NP = 10240
E = 320000
SUB = 125
J = 8
NC, NS = 2, 16
NW = NC * NS
CPW = E // (J * SUB * NW)
STRIPE = NP // NS


def _seg_sum_kernel(D, with_cnt):
    mesh = plsc.VectorSubcoreMesh(core_axis_name="c", subcore_axis_name="s")

    out_type = [jax.ShapeDtypeStruct((NC, NP, D), jnp.float32)]
    scratch = [
        pltpu.VMEM((J, SUB), jnp.int32),
        pltpu.VMEM((J, SUB), jnp.int32),
        pltpu.VMEM((J, SUB, D), jnp.float32),
        pltpu.VMEM_SHARED((NP, D), jnp.float32),
        pltpu.SemaphoreType.DMA,
    ]
    if with_cnt:
        out_type.append(jax.ShapeDtypeStruct((NC, NP), jnp.float32))
        scratch.insert(3, pltpu.VMEM((SUB,), jnp.float32))
        scratch.insert(5, pltpu.VMEM_SHARED((NP,), jnp.float32))

    def body(*refs):
        if with_cnt:
            (p_hbm, src_hbm, dst_hbm, z_hbm, ones_hbm, zc_hbm,
             part_hbm, cntp_hbm,
             src_v, dst_v, rows_v, ones_v, acc_sh, cnt_sh, sem) = refs
        else:
            (p_hbm, src_hbm, dst_hbm, z_hbm,
             part_hbm,
             src_v, dst_v, rows_v, acc_sh, sem) = refs
        c = lax.axis_index("c")
        s = lax.axis_index("s")

        pltpu.sync_copy(z_hbm.at[pl.ds(s * STRIPE, STRIPE)],
                        acc_sh.at[pl.ds(s * STRIPE, STRIPE)])
        if with_cnt:
            pltpu.sync_copy(ones_hbm, ones_v)

            @pl.when(s == 0)
            def _():
                pltpu.sync_copy(zc_hbm, cnt_sh)
        plsc.subcore_barrier()

        chunk0 = (c * NS + s) * CPW

        def step(i, carry):
            ch = chunk0 + i
            pltpu.sync_copy(src_hbm.at[ch], src_v)
            pltpu.sync_copy(dst_hbm.at[ch], dst_v)
            descs = [pltpu.async_copy(p_hbm.at[src_v.at[j]], rows_v.at[j],
                                      sem) for j in range(J)]
            for d in descs:
                d.wait()
            for j in range(J):
                pltpu.sync_copy(rows_v.at[j], acc_sh.at[dst_v.at[j]],
                                add=True)
            if with_cnt:
                for j in range(J):
                    pltpu.sync_copy(ones_v, cnt_sh.at[dst_v.at[j]], add=True)
            return carry

        lax.fori_loop(0, CPW, step, 0)
        plsc.subcore_barrier()

        pltpu.sync_copy(acc_sh.at[pl.ds(s * STRIPE, STRIPE)],
                        part_hbm.at[c, pl.ds(s * STRIPE, STRIPE)])
        if with_cnt:
            @pl.when(s == 0)
            def _():
                pltpu.sync_copy(cnt_sh, cntp_hbm.at[c])

    return pl.kernel(body, out_type=tuple(out_type), mesh=mesh,
                     scratch_types=tuple(scratch),
                     compiler_params=pltpu.CompilerParams(
                         use_tc_tiling_on_sc=False))


def _proj1_body(x_ref, wl_ref, wr_ref, b_ref, p_ref, r_ref):
    xv = x_ref[...]
    p_ref[...] = jnp.dot(xv, wl_ref[...], preferred_element_type=jnp.float32)
    r_ref[...] = (jnp.dot(xv, wr_ref[...], preferred_element_type=jnp.float32)
                  + b_ref[...])


def _mid_body(p0_ref, p1_ref, c0_ref, c1_ref, r1_ref, wl2_ref, wr2_ref,
              b2_ref, p2_ref, r2_ref, inv_ref):
    inv = 1.0 / jnp.maximum(c0_ref[...] + c1_ref[...], 1.0)
    h = jnp.maximum((p0_ref[...] + p1_ref[...]) * inv + r1_ref[...], 0.0)
    p2_ref[...] = jnp.dot(h, wl2_ref[...], preferred_element_type=jnp.float32)
    r2_ref[...] = (jnp.dot(h, wr2_ref[...], preferred_element_type=jnp.float32)
                   + b2_ref[...])
    inv_ref[...] = inv


def _out_body(q0_ref, q1_ref, inv_ref, r2_ref, o_ref):
    o_ref[...] = (q0_ref[...] + q1_ref[...]) * inv_ref[...] + r2_ref[...]


def kernel(x, edge_index, W_l1, W_r1, b1, W_l2, W_r2, b2):
    HID = W_l1.shape[1]
    OUT = W_l2.shape[1]
    D2 = 16

    src3 = edge_index[0].astype(jnp.int32).reshape(E // (J * SUB), J, SUB)
    dst3 = edge_index[1].astype(jnp.int32).reshape(E // (J * SUB), J, SUB)
    zeros1 = jnp.zeros((NP, HID), jnp.float32)
    zeros2 = jnp.zeros((NP, D2), jnp.float32)
    zcnt = jnp.zeros((NP,), jnp.float32)
    ones = jnp.ones((SUB,), jnp.float32)
    wl2p = jnp.zeros((HID, D2), jnp.float32).at[:, :OUT].set(W_l2)
    wr2p = jnp.zeros((HID, D2), jnp.float32).at[:, :OUT].set(W_r2)
    b2p = jnp.zeros((1, D2), jnp.float32).at[0, :OUT].set(b2)

    p1, r1 = pl.pallas_call(
        _proj1_body,
        out_shape=(jax.ShapeDtypeStruct((N, HID), jnp.float32),
                   jax.ShapeDtypeStruct((N, HID), jnp.float32)),
    )(x, W_l1, W_r1, b1.reshape(1, HID))

    part1, cntp = _seg_sum_kernel(HID, True)(p1, src3, dst3, zeros1, ones,
                                             zcnt)

    p2, r2, inv = pl.pallas_call(
        _mid_body,
        out_shape=(jax.ShapeDtypeStruct((N, D2), jnp.float32),
                   jax.ShapeDtypeStruct((N, D2), jnp.float32),
                   jax.ShapeDtypeStruct((N, 1), jnp.float32)),
    )(part1[0, :N], part1[1, :N], cntp[0, :N].reshape(N, 1),
      cntp[1, :N].reshape(N, 1), r1, wl2p, wr2p, b2p)

    (part2,) = _seg_sum_kernel(D2, False)(p2, src3, dst3, zeros2)

    outp = pl.pallas_call(
        _out_body,
        out_shape=jax.ShapeDtypeStruct((N, D2), jnp.float32),
    )(part2[0, :N], part2[1, :N], inv, r2)
    return outp[:, :OUT]


# speedup vs baseline: 14.7432x; 1.0644x over previous
import functools

import jax
import jax.numpy as jnp
from jax import lax
from jax.experimental import pallas as pl
from jax.experimental.pallas import tpu as pltpu
from jax.experimental.pallas import tpu_sc as plsc

N = 10000
NP = 10240
E = 320000
SUB = 125
J = 8
NC, NS = 2, 16
NW = NC * NS
CPW = E // (J * SUB * NW)
STRIPE = NP // NS


def _seg_sum_kernel(D, with_cnt):
    mesh = plsc.VectorSubcoreMesh(core_axis_name="c", subcore_axis_name="s")

    out_type = [jax.ShapeDtypeStruct((NC, NP, D), jnp.float32)]
    scratch = [
        pltpu.VMEM((J, SUB), jnp.int32),
        pltpu.VMEM((J, SUB), jnp.int32),
        pltpu.VMEM((J, SUB, D), jnp.float32),
        pltpu.VMEM_SHARED((NP, D), jnp.float32),
        pltpu.SemaphoreType.DMA,
    ]
    if with_cnt:
        out_type.append(jax.ShapeDtypeStruct((NC, NP), jnp.float32))
        scratch.insert(3, pltpu.VMEM((SUB,), jnp.float32))
        scratch.insert(5, pltpu.VMEM_SHARED((NP,), jnp.float32))

    def body(*refs):
        if with_cnt:
            (p_hbm, src_hbm, dst_hbm, z_hbm, ones_hbm, zc_hbm,
             part_hbm, cntp_hbm,
             src_v, dst_v, rows_v, ones_v, acc_sh, cnt_sh, sem) = refs
        else:
            (p_hbm, src_hbm, dst_hbm, z_hbm,
             part_hbm,
             src_v, dst_v, rows_v, acc_sh, sem) = refs
        c = lax.axis_index("c")
        s = lax.axis_index("s")

        pltpu.sync_copy(z_hbm.at[pl.ds(s * STRIPE, STRIPE)],
                        acc_sh.at[pl.ds(s * STRIPE, STRIPE)])
        if with_cnt:
            pltpu.sync_copy(ones_hbm, ones_v)

            @pl.when(s == 0)
            def _():
                pltpu.sync_copy(zc_hbm, cnt_sh)
        plsc.subcore_barrier()

        chunk0 = (c * NS + s) * CPW

        def step(i, carry):
            ch = chunk0 + i
            pltpu.sync_copy(src_hbm.at[ch], src_v)
            pltpu.sync_copy(dst_hbm.at[ch], dst_v)
            descs = [pltpu.async_copy(p_hbm.at[src_v.at[j]], rows_v.at[j],
                                      sem) for j in range(J)]
            for d in descs:
                d.wait()
            for j in range(J):
                pltpu.sync_copy(rows_v.at[j], acc_sh.at[dst_v.at[j]],
                                add=True)
            if with_cnt:
                for j in range(J):
                    pltpu.sync_copy(ones_v, cnt_sh.at[dst_v.at[j]], add=True)
            return carry

        lax.fori_loop(0, CPW, step, 0)
        plsc.subcore_barrier()

        pltpu.sync_copy(acc_sh.at[pl.ds(s * STRIPE, STRIPE)],
                        part_hbm.at[c, pl.ds(s * STRIPE, STRIPE)])
        if with_cnt:
            @pl.when(s == 0)
            def _():
                pltpu.sync_copy(cnt_sh, cntp_hbm.at[c])

    return pl.kernel(
        body,
        out_type=tuple(out_type),
        mesh=mesh,
        scratch_types=tuple(scratch),
        compiler_params=pltpu.CompilerParams(use_tc_tiling_on_sc=False))


def _proj1_body(x_ref, wl_ref, wr_ref, b_ref, p_ref, r_ref):
    xv = x_ref[...]
    p_ref[...] = jnp.dot(xv, wl_ref[...], preferred_element_type=jnp.float32)
    r_ref[...] = (jnp.dot(xv, wr_ref[...], preferred_element_type=jnp.float32)
                  + b_ref[...])


def _mid_body(part_ref, c0_ref, c1_ref, r1_ref, wl2_ref, wr2_ref, b2_ref,
              p2_ref, r2_ref, inv_ref):
    a = part_ref[0, :N] + part_ref[1, :N]
    inv = 1.0 / jnp.maximum(c0_ref[...] + c1_ref[...], 1.0)
    h = jnp.maximum(a * inv + r1_ref[...], 0.0)
    p2_ref[...] = jnp.dot(h, wl2_ref[...], preferred_element_type=jnp.float32)
    r2_ref[...] = (jnp.dot(h, wr2_ref[...], preferred_element_type=jnp.float32)
                   + b2_ref[...])
    inv_ref[...] = inv


def _out_body(part_ref, inv_ref, r2_ref, o_ref):
    o_ref[...] = ((part_ref[0, :N] + part_ref[1, :N]) * inv_ref[...]
                  + r2_ref[...])


def kernel(x, edge_index, W_l1, W_r1, b1, W_l2, W_r2, b2):
    HID = W_l1.shape[1]
    OUT = W_l2.shape[1]
    D2 = 16

    src3 = edge_index[0].astype(jnp.int32).reshape(E // (J * SUB), J, SUB)
    dst3 = edge_index[1].astype(jnp.int32).reshape(E // (J * SUB), J, SUB)
    zeros1 = jnp.zeros((NP, HID), jnp.float32)
    zeros2 = jnp.zeros((NP, D2), jnp.float32)
    zcnt = jnp.zeros((NP,), jnp.float32)
    ones = jnp.ones((SUB,), jnp.float32)
    wl2p = jnp.zeros((HID, D2), jnp.float32).at[:, :OUT].set(W_l2)
    wr2p = jnp.zeros((HID, D2), jnp.float32).at[:, :OUT].set(W_r2)
    b2p = jnp.zeros((1, D2), jnp.float32).at[0, :OUT].set(b2)

    p1, r1 = pl.pallas_call(
        _proj1_body,
        out_shape=(jax.ShapeDtypeStruct((N, HID), jnp.float32),
                   jax.ShapeDtypeStruct((N, HID), jnp.float32)),
    )(x, W_l1, W_r1, b1.reshape(1, HID))

    part1, cntp = _seg_sum_kernel(HID, True)(p1, src3, dst3, zeros1, ones,
                                             zcnt)

    p2, r2, inv = pl.pallas_call(
        _mid_body,
        out_shape=(jax.ShapeDtypeStruct((N, D2), jnp.float32),
                   jax.ShapeDtypeStruct((N, D2), jnp.float32),
                   jax.ShapeDtypeStruct((N, 1), jnp.float32)),
    )(part1, cntp[0, :N].reshape(N, 1), cntp[1, :N].reshape(N, 1),
      r1, wl2p, wr2p, b2p)

    (part2,) = _seg_sum_kernel(D2, False)(p2, src3, dst3, zeros2)

    outp = pl.pallas_call(
        _out_body,
        out_shape=jax.ShapeDtypeStruct((N, D2), jnp.float32),
    )(part2, inv, r2)
    return outp[:, :OUT]


# speedup vs baseline: 17.4804x; 1.1857x over previous
import functools

import jax
import jax.numpy as jnp
from jax import lax
from jax.experimental import pallas as pl
from jax.experimental.pallas import tpu as pltpu
from jax.experimental.pallas import tpu_sc as plsc

N = 10000
NP = 10240
E = 320000
SUB = 125
NC, NS = 2, 16
NW = NC * NS
STRIPE = NP // NS


def _seg_sum_kernel(D, J, with_cnt):
    mesh = plsc.VectorSubcoreMesh(core_axis_name="c", subcore_axis_name="s")
    cpw = E // (J * SUB * NW)
    assert cpw % 2 == 0

    out_type = [jax.ShapeDtypeStruct((NC, NP, D), jnp.float32)]
    scratch = [
        pltpu.VMEM((2, J, SUB), jnp.int32),
        pltpu.VMEM((2, J, SUB), jnp.int32),
        pltpu.VMEM((2, J, SUB, D), jnp.float32),
        pltpu.VMEM_SHARED((NP, D), jnp.float32),
        pltpu.SemaphoreType.DMA,
        pltpu.SemaphoreType.DMA,
    ]
    if with_cnt:
        out_type.append(jax.ShapeDtypeStruct((NC, NP), jnp.float32))
        scratch.insert(3, pltpu.VMEM((SUB,), jnp.float32))
        scratch.insert(5, pltpu.VMEM_SHARED((NP,), jnp.float32))

    def body(*refs):
        if with_cnt:
            (p_hbm, src_hbm, dst_hbm, z_hbm, ones_hbm, zc_hbm,
             part_hbm, cntp_hbm,
             src_v, dst_v, rows_v, ones_v, acc_sh, cnt_sh,
             sem_a, sem_b) = refs
        else:
            (p_hbm, src_hbm, dst_hbm, z_hbm,
             part_hbm,
             src_v, dst_v, rows_v, acc_sh, sem_a, sem_b) = refs
        c = lax.axis_index("c")
        s = lax.axis_index("s")
        sems = (sem_a, sem_b)

        pltpu.sync_copy(z_hbm.at[pl.ds(s * STRIPE, STRIPE)],
                        acc_sh.at[pl.ds(s * STRIPE, STRIPE)])
        if with_cnt:
            pltpu.sync_copy(ones_hbm, ones_v)

            @pl.when(s == 0)
            def _():
                pltpu.sync_copy(zc_hbm, cnt_sh)
        plsc.subcore_barrier()

        chunk0 = (c * NS + s) * cpw

        def fire(ch, b):
            pltpu.sync_copy(src_hbm.at[ch], src_v.at[b])
            pltpu.sync_copy(dst_hbm.at[ch], dst_v.at[b])
            for j in range(J):
                pltpu.async_copy(p_hbm.at[src_v.at[b, j]], rows_v.at[b, j],
                                 sems[b])

        def drain_scatter(b):
            for j in range(J):
                pltpu.make_async_copy(p_hbm.at[src_v.at[b, j]],
                                      rows_v.at[b, j], sems[b]).wait()
            for j in range(J):
                pltpu.sync_copy(rows_v.at[b, j], acc_sh.at[dst_v.at[b, j]],
                                add=True)
            if with_cnt:
                for j in range(J):
                    pltpu.sync_copy(ones_v, cnt_sh.at[dst_v.at[b, j]],
                                    add=True)

        fire(chunk0, 0)

        def step(i, carry):
            fire(chunk0 + 2 * i + 1, 1)
            drain_scatter(0)

            @pl.when(i < cpw // 2 - 1)
            def _():
                fire(chunk0 + 2 * i + 2, 0)
            drain_scatter(1)
            return carry

        lax.fori_loop(0, cpw // 2, step, 0)
        plsc.subcore_barrier()

        pltpu.sync_copy(acc_sh.at[pl.ds(s * STRIPE, STRIPE)],
                        part_hbm.at[c, pl.ds(s * STRIPE, STRIPE)])
        if with_cnt:
            @pl.when(s == 0)
            def _():
                pltpu.sync_copy(cnt_sh, cntp_hbm.at[c])

    return pl.kernel(
        body,
        out_type=tuple(out_type),
        mesh=mesh,
        scratch_types=tuple(scratch),
        compiler_params=pltpu.CompilerParams(use_tc_tiling_on_sc=False))


def _proj1_body(x_ref, wl_ref, wr_ref, b_ref, p_ref, r_ref):
    xv = x_ref[...]
    p_ref[...] = jnp.dot(xv, wl_ref[...], preferred_element_type=jnp.float32)
    r_ref[...] = (jnp.dot(xv, wr_ref[...], preferred_element_type=jnp.float32)
                  + b_ref[...])


def _mid_body(part_ref, c0_ref, c1_ref, r1_ref, wl2_ref, wr2_ref, b2_ref,
              p2_ref, r2_ref, inv_ref):
    a = part_ref[0, :N] + part_ref[1, :N]
    inv = 1.0 / jnp.maximum(c0_ref[...] + c1_ref[...], 1.0)
    h = jnp.maximum(a * inv + r1_ref[...], 0.0)
    p2_ref[...] = jnp.dot(h, wl2_ref[...], preferred_element_type=jnp.float32)
    r2_ref[...] = (jnp.dot(h, wr2_ref[...], preferred_element_type=jnp.float32)
                   + b2_ref[...])
    inv_ref[...] = inv


def _out_body(part_ref, inv_ref, r2_ref, o_ref):
    o_ref[...] = ((part_ref[0, :N] + part_ref[1, :N]) * inv_ref[...]
                  + r2_ref[...])


def kernel(x, edge_index, W_l1, W_r1, b1, W_l2, W_r2, b2):
    HID = W_l1.shape[1]
    OUT = W_l2.shape[1]
    D2 = 16

    J1, J2 = 4, 8
    src = edge_index[0].astype(jnp.int32)
    dst = edge_index[1].astype(jnp.int32)
    src3a = src.reshape(E // (J1 * SUB), J1, SUB)
    dst3a = dst.reshape(E // (J1 * SUB), J1, SUB)
    src3b = src.reshape(E // (J2 * SUB), J2, SUB)
    dst3b = dst.reshape(E // (J2 * SUB), J2, SUB)
    zeros1 = jnp.zeros((NP, HID), jnp.float32)
    zeros2 = jnp.zeros((NP, D2), jnp.float32)
    zcnt = jnp.zeros((NP,), jnp.float32)
    ones = jnp.ones((SUB,), jnp.float32)
    wl2p = jnp.zeros((HID, D2), jnp.float32).at[:, :OUT].set(W_l2)
    wr2p = jnp.zeros((HID, D2), jnp.float32).at[:, :OUT].set(W_r2)
    b2p = jnp.zeros((1, D2), jnp.float32).at[0, :OUT].set(b2)

    p1, r1 = pl.pallas_call(
        _proj1_body,
        out_shape=(jax.ShapeDtypeStruct((N, HID), jnp.float32),
                   jax.ShapeDtypeStruct((N, HID), jnp.float32)),
    )(x, W_l1, W_r1, b1.reshape(1, HID))

    part1, cntp = _seg_sum_kernel(HID, J1, True)(p1, src3a, dst3a, zeros1,
                                                 ones, zcnt)

    p2, r2, inv = pl.pallas_call(
        _mid_body,
        out_shape=(jax.ShapeDtypeStruct((N, D2), jnp.float32),
                   jax.ShapeDtypeStruct((N, D2), jnp.float32),
                   jax.ShapeDtypeStruct((N, 1), jnp.float32)),
    )(part1, cntp[0, :N].reshape(N, 1), cntp[1, :N].reshape(N, 1),
      r1, wl2p, wr2p, b2p)

    (part2,) = _seg_sum_kernel(D2, J2, False)(p2, src3b, dst3b, zeros2)

    outp = pl.pallas_call(
        _out_body,
        out_shape=jax.ShapeDtypeStruct((N, D2), jnp.float32),
    )(part2, inv, r2)
    return outp[:, :OUT]


# speedup vs baseline: 17.9628x; 1.0276x over previous
import functools

import jax
import jax.numpy as jnp
from jax import lax
from jax.experimental import pallas as pl
from jax.experimental.pallas import tpu as pltpu
from jax.experimental.pallas import tpu_sc as plsc

N = 10000
NP = 10240
E = 320000
SUB = 125
NC, NS = 2, 16
NW = NC * NS
STRIPE = NP // NS


def _seg_sum_kernel(D, J, with_cnt):
    mesh = plsc.VectorSubcoreMesh(core_axis_name="c", subcore_axis_name="s")
    cpw = E // (J * SUB * NW)
    assert cpw % 2 == 0

    ZR = 128

    out_type = [jax.ShapeDtypeStruct((NC, NP, D), jnp.float32)]
    scratch = [
        pltpu.VMEM((2, J, SUB), jnp.int32),
        pltpu.VMEM((2, J, SUB), jnp.int32),
        pltpu.VMEM((2, J, SUB, D), jnp.float32),
        pltpu.VMEM((ZR, D), jnp.float32),
        pltpu.VMEM_SHARED((NP, D), jnp.float32),
        pltpu.SemaphoreType.DMA,
        pltpu.SemaphoreType.DMA,
    ]
    if with_cnt:
        out_type.append(jax.ShapeDtypeStruct((NC, NP), jnp.float32))
        scratch.insert(4, pltpu.VMEM((SUB,), jnp.float32))
        scratch.insert(6, pltpu.VMEM_SHARED((NP,), jnp.float32))

    def body(*refs):
        if with_cnt:
            (p_hbm, src_hbm, dst_hbm,
             part_hbm, cntp_hbm,
             src_v, dst_v, rows_v, zero_v, ones_v, acc_sh, cnt_sh,
             sem_a, sem_b) = refs
        else:
            (p_hbm, src_hbm, dst_hbm,
             part_hbm,
             src_v, dst_v, rows_v, zero_v, acc_sh, sem_a, sem_b) = refs
        c = lax.axis_index("c")
        s = lax.axis_index("s")
        sems = (sem_a, sem_b)

        zv = jnp.zeros((16,), jnp.float32)

        def zfill(r, carry):
            for k in range(D // 16):
                zero_v[r, pl.ds(k * 16, 16)] = zv
            return carry

        lax.fori_loop(0, ZR, zfill, 0)
        for t in range(STRIPE // ZR):
            pltpu.sync_copy(zero_v,
                            acc_sh.at[pl.ds(s * STRIPE + t * ZR, ZR)])
        if with_cnt:
            ov = jnp.ones((16,), jnp.float32)
            for o in (0, 16, 32, 48, 64, 80, 96, SUB - 16):
                ones_v[pl.ds(o, 16)] = ov
            for t in range(STRIPE // D):
                pltpu.sync_copy(zero_v.at[0],
                                cnt_sh.at[pl.ds(s * STRIPE + t * D, D)])
        plsc.subcore_barrier()

        chunk0 = (c * NS + s) * cpw

        def fire(ch, b):
            pltpu.sync_copy(src_hbm.at[ch], src_v.at[b])
            pltpu.sync_copy(dst_hbm.at[ch], dst_v.at[b])
            for j in range(J):
                pltpu.async_copy(p_hbm.at[src_v.at[b, j]], rows_v.at[b, j],
                                 sems[b])

        def drain_scatter(b):
            for j in range(J):
                pltpu.make_async_copy(p_hbm.at[src_v.at[b, j]],
                                      rows_v.at[b, j], sems[b]).wait()
            for j in range(J):
                pltpu.sync_copy(rows_v.at[b, j], acc_sh.at[dst_v.at[b, j]],
                                add=True)
            if with_cnt:
                for j in range(J):
                    pltpu.sync_copy(ones_v, cnt_sh.at[dst_v.at[b, j]],
                                    add=True)

        fire(chunk0, 0)

        def step(i, carry):
            fire(chunk0 + 2 * i + 1, 1)
            drain_scatter(0)

            @pl.when(i < cpw // 2 - 1)
            def _():
                fire(chunk0 + 2 * i + 2, 0)
            drain_scatter(1)
            return carry

        lax.fori_loop(0, cpw // 2, step, 0)
        plsc.subcore_barrier()

        pltpu.sync_copy(acc_sh.at[pl.ds(s * STRIPE, STRIPE)],
                        part_hbm.at[c, pl.ds(s * STRIPE, STRIPE)])
        if with_cnt:
            @pl.when(s == 0)
            def _():
                pltpu.sync_copy(cnt_sh, cntp_hbm.at[c])

    return pl.kernel(
        body,
        out_type=tuple(out_type),
        mesh=mesh,
        scratch_types=tuple(scratch),
        compiler_params=pltpu.CompilerParams(use_tc_tiling_on_sc=False))


def _proj1_body(x_ref, wl_ref, wr_ref, b_ref, p_ref, r_ref):
    xv = x_ref[...]
    p_ref[...] = jnp.dot(xv, wl_ref[...], preferred_element_type=jnp.float32)
    r_ref[...] = (jnp.dot(xv, wr_ref[...], preferred_element_type=jnp.float32)
                  + b_ref[...])


def _mid_body(part_ref, c0_ref, c1_ref, r1_ref, wl2_ref, wr2_ref, b2_ref,
              p2_ref, r2_ref, inv_ref):
    a = part_ref[0, :N] + part_ref[1, :N]
    inv = 1.0 / jnp.maximum(c0_ref[...] + c1_ref[...], 1.0)
    h = jnp.maximum(a * inv + r1_ref[...], 0.0)
    p2_ref[...] = jnp.dot(h, wl2_ref[...], preferred_element_type=jnp.float32)
    r2_ref[...] = (jnp.dot(h, wr2_ref[...], preferred_element_type=jnp.float32)
                   + b2_ref[...])
    inv_ref[...] = inv


def _out_body(part_ref, inv_ref, r2_ref, o_ref):
    o_ref[...] = ((part_ref[0, :N] + part_ref[1, :N]) * inv_ref[...]
                  + r2_ref[...])


def kernel(x, edge_index, W_l1, W_r1, b1, W_l2, W_r2, b2):
    HID = W_l1.shape[1]
    OUT = W_l2.shape[1]
    D2 = 16

    J1, J2 = 4, 8
    src = edge_index[0].astype(jnp.int32)
    dst = edge_index[1].astype(jnp.int32)
    src3a = src.reshape(E // (J1 * SUB), J1, SUB)
    dst3a = dst.reshape(E // (J1 * SUB), J1, SUB)
    src3b = src.reshape(E // (J2 * SUB), J2, SUB)
    dst3b = dst.reshape(E // (J2 * SUB), J2, SUB)
    wl2p = jnp.zeros((HID, D2), jnp.float32).at[:, :OUT].set(W_l2)
    wr2p = jnp.zeros((HID, D2), jnp.float32).at[:, :OUT].set(W_r2)
    b2p = jnp.zeros((1, D2), jnp.float32).at[0, :OUT].set(b2)

    p1, r1 = pl.pallas_call(
        _proj1_body,
        out_shape=(jax.ShapeDtypeStruct((N, HID), jnp.float32),
                   jax.ShapeDtypeStruct((N, HID), jnp.float32)),
    )(x, W_l1, W_r1, b1.reshape(1, HID))

    part1, cntp = _seg_sum_kernel(HID, J1, True)(p1, src3a, dst3a)

    p2, r2, inv = pl.pallas_call(
        _mid_body,
        out_shape=(jax.ShapeDtypeStruct((N, D2), jnp.float32),
                   jax.ShapeDtypeStruct((N, D2), jnp.float32),
                   jax.ShapeDtypeStruct((N, 1), jnp.float32)),
    )(part1, cntp[0, :N].reshape(N, 1), cntp[1, :N].reshape(N, 1),
      r1, wl2p, wr2p, b2p)

    (part2,) = _seg_sum_kernel(D2, J2, False)(p2, src3b, dst3b)

    outp = pl.pallas_call(
        _out_body,
        out_shape=jax.ShapeDtypeStruct((N, D2), jnp.float32),
    )(part2, inv, r2)
    return outp[:, :OUT]


# speedup vs baseline: 18.0894x; 1.0071x over previous
import functools

import jax
import jax.numpy as jnp
from jax import lax
from jax.experimental import pallas as pl
from jax.experimental.pallas import tpu as pltpu
from jax.experimental.pallas import tpu_sc as plsc

N = 10000
NP = 10240
E = 320000
SUB = 125
NC, NS = 2, 16
NW = NC * NS
STRIPE = NP // NS


def _seg_sum_kernel(D, J, with_cnt):
    mesh = plsc.VectorSubcoreMesh(core_axis_name="c", subcore_axis_name="s")
    cpw = E // (J * SUB * NW)
    assert cpw % 2 == 0

    ZR = 128

    out_type = [jax.ShapeDtypeStruct((NC, NP, D), jnp.float32)]
    scratch = [
        pltpu.VMEM((2, J, SUB), jnp.int32),
        pltpu.VMEM((2, J, SUB), jnp.int32),
        pltpu.VMEM((2, J, SUB, D), jnp.float32),
        pltpu.VMEM((ZR, D), jnp.float32),
        pltpu.VMEM_SHARED((NP, D), jnp.float32),
        pltpu.SemaphoreType.DMA,
        pltpu.SemaphoreType.DMA,
    ]
    if with_cnt:
        out_type.append(jax.ShapeDtypeStruct((NC, NP), jnp.float32))
        scratch.insert(4, pltpu.VMEM((SUB,), jnp.float32))
        scratch.insert(6, pltpu.VMEM_SHARED((NP,), jnp.float32))

    def body(*refs):
        if with_cnt:
            (p_hbm, src_hbm, dst_hbm,
             part_hbm, cntp_hbm,
             src_v, dst_v, rows_v, zero_v, ones_v, acc_sh, cnt_sh,
             sem_a, sem_b) = refs
        else:
            (p_hbm, src_hbm, dst_hbm,
             part_hbm,
             src_v, dst_v, rows_v, zero_v, acc_sh, sem_a, sem_b) = refs
        c = lax.axis_index("c")
        s = lax.axis_index("s")
        sems = (sem_a, sem_b)

        zv = jnp.zeros((16,), jnp.float32)

        def zfill(r, carry):
            for k in range(D // 16):
                zero_v[r, pl.ds(k * 16, 16)] = zv
            return carry

        lax.fori_loop(0, ZR, zfill, 0)
        for t in range(STRIPE // ZR):
            pltpu.sync_copy(zero_v,
                            acc_sh.at[pl.ds(s * STRIPE + t * ZR, ZR)])
        if with_cnt:
            ov = jnp.ones((16,), jnp.float32)
            for o in (0, 16, 32, 48, 64, 80, 96, SUB - 16):
                ones_v[pl.ds(o, 16)] = ov
            for t in range(STRIPE // D):
                pltpu.sync_copy(zero_v.at[0],
                                cnt_sh.at[pl.ds(s * STRIPE + t * D, D)])
        plsc.subcore_barrier()

        chunk0 = (c * NS + s) * cpw

        def fire(ch, b):
            pltpu.sync_copy(src_hbm.at[ch], src_v.at[b])
            pltpu.sync_copy(dst_hbm.at[ch], dst_v.at[b])
            for j in range(J):
                pltpu.async_copy(p_hbm.at[src_v.at[b, j]], rows_v.at[b, j],
                                 sems[b])

        def drain_scatter(b):
            for j in range(J):
                pltpu.make_async_copy(p_hbm.at[src_v.at[b, j]],
                                      rows_v.at[b, j], sems[b]).wait()
            for j in range(J):
                pltpu.sync_copy(rows_v.at[b, j], acc_sh.at[dst_v.at[b, j]],
                                add=True)
            if with_cnt:
                for j in range(J):
                    pltpu.sync_copy(ones_v, cnt_sh.at[dst_v.at[b, j]],
                                    add=True)

        fire(chunk0, 0)

        def step(i, carry):
            fire(chunk0 + 2 * i + 1, 1)
            drain_scatter(0)

            @pl.when(i < cpw // 2 - 1)
            def _():
                fire(chunk0 + 2 * i + 2, 0)
            drain_scatter(1)
            return carry

        lax.fori_loop(0, cpw // 2, step, 0)
        plsc.subcore_barrier()

        pltpu.sync_copy(acc_sh.at[pl.ds(s * STRIPE, STRIPE)],
                        part_hbm.at[c, pl.ds(s * STRIPE, STRIPE)])
        if with_cnt:
            @pl.when(s == 0)
            def _():
                pltpu.sync_copy(cnt_sh, cntp_hbm.at[c])

    return pl.kernel(
        body,
        out_type=tuple(out_type),
        mesh=mesh,
        scratch_types=tuple(scratch),
        compiler_params=pltpu.CompilerParams(use_tc_tiling_on_sc=False))


def _proj1_body(x_ref, wl_ref, wr_ref, b_ref, p_ref, r_ref):
    xv = x_ref[...]
    p_ref[...] = jnp.dot(xv, wl_ref[...], preferred_element_type=jnp.float32)
    r_ref[...] = (jnp.dot(xv, wr_ref[...], preferred_element_type=jnp.float32)
                  + b_ref[...])


def _mid_body(part_ref, c0_ref, c1_ref, r1_ref, wl2_ref, wr2_ref, b2_ref,
              p2_ref, r2_ref, inv_ref):
    a = part_ref[0, :N] + part_ref[1, :N]
    inv = 1.0 / jnp.maximum(c0_ref[...] + c1_ref[...], 1.0)
    h = jnp.maximum(a * inv + r1_ref[...], 0.0)
    p2_ref[...] = jnp.dot(h, wl2_ref[...], preferred_element_type=jnp.float32)
    r2_ref[...] = (jnp.dot(h, wr2_ref[...], preferred_element_type=jnp.float32)
                   + b2_ref[...])
    inv_ref[...] = inv


def _out_body(part_ref, inv_ref, r2_ref, o_ref):
    o_ref[...] = ((part_ref[0, :N] + part_ref[1, :N]) * inv_ref[...]
                  + r2_ref[...])


def kernel(x, edge_index, W_l1, W_r1, b1, W_l2, W_r2, b2):
    HID = W_l1.shape[1]
    OUT = W_l2.shape[1]
    D2 = 16

    J1, J2 = 4, 10
    src = edge_index[0].astype(jnp.int32)
    dst = edge_index[1].astype(jnp.int32)
    src3a = src.reshape(E // (J1 * SUB), J1, SUB)
    dst3a = dst.reshape(E // (J1 * SUB), J1, SUB)
    src3b = src.reshape(E // (J2 * SUB), J2, SUB)
    dst3b = dst.reshape(E // (J2 * SUB), J2, SUB)
    wl2p = jnp.zeros((HID, D2), jnp.float32).at[:, :OUT].set(W_l2)
    wr2p = jnp.zeros((HID, D2), jnp.float32).at[:, :OUT].set(W_r2)
    b2p = jnp.zeros((1, D2), jnp.float32).at[0, :OUT].set(b2)

    p1, r1 = pl.pallas_call(
        _proj1_body,
        out_shape=(jax.ShapeDtypeStruct((N, HID), jnp.float32),
                   jax.ShapeDtypeStruct((N, HID), jnp.float32)),
    )(x, W_l1, W_r1, b1.reshape(1, HID))

    part1, cntp = _seg_sum_kernel(HID, J1, True)(p1, src3a, dst3a)

    p2, r2, inv = pl.pallas_call(
        _mid_body,
        out_shape=(jax.ShapeDtypeStruct((N, D2), jnp.float32),
                   jax.ShapeDtypeStruct((N, D2), jnp.float32),
                   jax.ShapeDtypeStruct((N, 1), jnp.float32)),
    )(part1, cntp[0, :N].reshape(N, 1), cntp[1, :N].reshape(N, 1),
      r1, wl2p, wr2p, b2p)

    (part2,) = _seg_sum_kernel(D2, J2, False)(p2, src3b, dst3b)

    outp = pl.pallas_call(
        _out_body,
        out_shape=jax.ShapeDtypeStruct((N, D2), jnp.float32),
    )(part2, inv, r2)
    return outp[:, :OUT]


# speedup vs baseline: 18.5753x; 1.0269x over previous
import functools

import jax
import jax.numpy as jnp
from jax import lax
from jax.experimental import pallas as pl
from jax.experimental.pallas import tpu as pltpu
from jax.experimental.pallas import tpu_sc as plsc

N = 10000
NP = 10240
E = 320000
SUB = 125
NC, NS = 2, 16
NW = NC * NS
STRIPE = NP // NS


def _seg_sum_kernel(D, J, with_cnt):
    mesh = plsc.VectorSubcoreMesh(core_axis_name="c", subcore_axis_name="s")
    cpw = E // (J * SUB * NW)
    assert cpw % 2 == 0

    ZR = 128

    out_type = [jax.ShapeDtypeStruct((NC, NP, D), jnp.float32)]
    scratch = [
        pltpu.VMEM((2, J, SUB), jnp.int32),
        pltpu.VMEM((2, J, SUB), jnp.int32),
        pltpu.VMEM((2, J, SUB, D), jnp.float32),
        pltpu.VMEM((ZR, D), jnp.float32),
        pltpu.VMEM_SHARED((NP, D), jnp.float32),
        pltpu.SemaphoreType.DMA,
        pltpu.SemaphoreType.DMA,
    ]
    if with_cnt:
        out_type.append(jax.ShapeDtypeStruct((NC, NP), jnp.float32))
        scratch.insert(4, pltpu.VMEM((SUB,), jnp.float32))
        scratch.insert(6, pltpu.VMEM_SHARED((NP,), jnp.float32))

    def body(*refs):
        if with_cnt:
            (p_hbm, src_hbm, dst_hbm,
             part_hbm, cntp_hbm,
             src_v, dst_v, rows_v, zero_v, ones_v, acc_sh, cnt_sh,
             sem_a, sem_b) = refs
        else:
            (p_hbm, src_hbm, dst_hbm,
             part_hbm,
             src_v, dst_v, rows_v, zero_v, acc_sh, sem_a, sem_b) = refs
        c = lax.axis_index("c")
        s = lax.axis_index("s")
        sems = (sem_a, sem_b)

        zv = jnp.zeros((16,), jnp.float32)

        def zfill(r, carry):
            for k in range(D // 16):
                zero_v[r, pl.ds(k * 16, 16)] = zv
            return carry

        lax.fori_loop(0, ZR, zfill, 0)
        for t in range(STRIPE // ZR):
            pltpu.sync_copy(zero_v,
                            acc_sh.at[pl.ds(s * STRIPE + t * ZR, ZR)])
        if with_cnt:
            ov = jnp.ones((16,), jnp.float32)
            for o in (0, 16, 32, 48, 64, 80, 96, SUB - 16):
                ones_v[pl.ds(o, 16)] = ov
            for t in range(STRIPE // D):
                pltpu.sync_copy(zero_v.at[0],
                                cnt_sh.at[pl.ds(s * STRIPE + t * D, D)])
        plsc.subcore_barrier()

        chunk0 = (c * NS + s) * cpw

        def fire(ch, b):
            pltpu.sync_copy(src_hbm.at[ch], src_v.at[b])
            pltpu.sync_copy(dst_hbm.at[ch], dst_v.at[b])
            for j in range(J):
                pltpu.async_copy(p_hbm.at[src_v.at[b, j]], rows_v.at[b, j],
                                 sems[b])

        def drain_scatter(b):
            for j in range(J):
                pltpu.make_async_copy(p_hbm.at[src_v.at[b, j]],
                                      rows_v.at[b, j], sems[b]).wait()
            for j in range(J):
                pltpu.sync_copy(rows_v.at[b, j], acc_sh.at[dst_v.at[b, j]],
                                add=True)
            if with_cnt:
                for j in range(J):
                    pltpu.sync_copy(ones_v, cnt_sh.at[dst_v.at[b, j]],
                                    add=True)

        fire(chunk0, 0)

        def step(i, carry):
            fire(chunk0 + 2 * i + 1, 1)
            drain_scatter(0)

            @pl.when(i < cpw // 2 - 1)
            def _():
                fire(chunk0 + 2 * i + 2, 0)
            drain_scatter(1)
            return carry

        lax.fori_loop(0, cpw // 2, step, 0)
        plsc.subcore_barrier()

        pltpu.sync_copy(acc_sh.at[pl.ds(s * STRIPE, STRIPE)],
                        part_hbm.at[c, pl.ds(s * STRIPE, STRIPE)])
        if with_cnt:
            @pl.when(s == 0)
            def _():
                pltpu.sync_copy(cnt_sh, cntp_hbm.at[c])

    return pl.kernel(
        body,
        out_type=tuple(out_type),
        mesh=mesh,
        scratch_types=tuple(scratch),
        compiler_params=pltpu.CompilerParams(use_tc_tiling_on_sc=False))


def _proj1_body(x_ref, wl_ref, wr_ref, b_ref, p_ref, r_ref):
    xv = x_ref[...]
    p_ref[...] = jnp.dot(xv, wl_ref[...], preferred_element_type=jnp.float32)
    r_ref[...] = (jnp.dot(xv, wr_ref[...], preferred_element_type=jnp.float32)
                  + b_ref[...])


def _mid_body(part_ref, cnt_ref, r1_ref, wl2_ref, wr2_ref, b2_ref,
              p2_ref, r2_ref, inv_ref):
    a = part_ref[0, :N] + part_ref[1, :N]
    cnt = cnt_ref[0, :N] + cnt_ref[1, :N]
    inv = (1.0 / jnp.maximum(cnt, 1.0))[:, None]
    h = jnp.maximum(a * inv + r1_ref[...], 0.0)
    p2 = jnp.dot(h, wl2_ref[...], preferred_element_type=jnp.float32)
    p2_ref[...] = jnp.concatenate(
        [p2, jnp.zeros((N, 16 - p2.shape[1]), jnp.float32)], axis=1)
    r2_ref[...] = (jnp.dot(h, wr2_ref[...], preferred_element_type=jnp.float32)
                   + b2_ref[...])
    inv_ref[...] = inv


def _out_body(part_ref, inv_ref, r2_ref, o_ref):
    q = part_ref[0, :N, :2] + part_ref[1, :N, :2]
    o_ref[...] = q * inv_ref[...] + r2_ref[...]


def kernel(x, edge_index, W_l1, W_r1, b1, W_l2, W_r2, b2):
    HID = W_l1.shape[1]
    OUT = W_l2.shape[1]
    D2 = 16

    J1, J2 = 4, 10
    src = edge_index[0].astype(jnp.int32)
    dst = edge_index[1].astype(jnp.int32)
    src3a = src.reshape(E // (J1 * SUB), J1, SUB)
    dst3a = dst.reshape(E // (J1 * SUB), J1, SUB)
    src3b = src.reshape(E // (J2 * SUB), J2, SUB)
    dst3b = dst.reshape(E // (J2 * SUB), J2, SUB)

    p1, r1 = pl.pallas_call(
        _proj1_body,
        out_shape=(jax.ShapeDtypeStruct((N, HID), jnp.float32),
                   jax.ShapeDtypeStruct((N, HID), jnp.float32)),
    )(x, W_l1, W_r1, b1)

    part1, cntp = _seg_sum_kernel(HID, J1, True)(p1, src3a, dst3a)

    p2, r2, inv = pl.pallas_call(
        _mid_body,
        out_shape=(jax.ShapeDtypeStruct((N, D2), jnp.float32),
                   jax.ShapeDtypeStruct((N, OUT), jnp.float32),
                   jax.ShapeDtypeStruct((N, 1), jnp.float32)),
    )(part1, cntp, r1, W_l2, W_r2, b2)

    (part2,) = _seg_sum_kernel(D2, J2, False)(p2, src3b, dst3b)

    return pl.pallas_call(
        _out_body,
        out_shape=jax.ShapeDtypeStruct((N, OUT), jnp.float32),
    )(part2, inv, r2)


# speedup vs baseline: 19.9239x; 1.0726x over previous
import functools

import jax
import jax.numpy as jnp
from jax import lax
from jax.experimental import pallas as pl
from jax.experimental.pallas import tpu as pltpu
from jax.experimental.pallas import tpu_sc as plsc

N = 10000
NP = 10240
E = 320000
SUB = 125
NC, NS = 2, 16
NW = NC * NS
STRIPE = NP // NS


def _seg_sum_kernel(D, J, with_cnt):
    mesh = plsc.VectorSubcoreMesh(core_axis_name="c", subcore_axis_name="s")
    cpw = E // (J * SUB * NW)
    assert cpw % 2 == 0
    RW = cpw * J

    ZR = 64

    out_type = [jax.ShapeDtypeStruct((NC, NP, D), jnp.float32)]
    scratch = [
        pltpu.VMEM((RW, SUB), jnp.int32),
        pltpu.VMEM((RW, SUB), jnp.int32),
        pltpu.VMEM((2, J, SUB, D), jnp.float32),
        pltpu.VMEM((ZR, D), jnp.float32),
        pltpu.VMEM_SHARED((NP, D), jnp.float32),
        pltpu.SemaphoreType.DMA,
        pltpu.SemaphoreType.DMA,
    ]
    if with_cnt:
        out_type.append(jax.ShapeDtypeStruct((NC, NP), jnp.float32))
        scratch.insert(4, pltpu.VMEM((SUB,), jnp.float32))
        scratch.insert(6, pltpu.VMEM_SHARED((NP,), jnp.float32))

    def body(*refs):
        if with_cnt:
            (p_hbm, src_hbm, dst_hbm,
             part_hbm, cntp_hbm,
             src_v, dst_v, rows_v, zero_v, ones_v, acc_sh, cnt_sh,
             sem_a, sem_b) = refs
        else:
            (p_hbm, src_hbm, dst_hbm,
             part_hbm,
             src_v, dst_v, rows_v, zero_v, acc_sh, sem_a, sem_b) = refs
        c = lax.axis_index("c")
        s = lax.axis_index("s")
        sems = (sem_a, sem_b)

        row0 = (c * NS + s) * RW
        pltpu.sync_copy(src_hbm.at[pl.ds(row0, RW)], src_v)
        pltpu.sync_copy(dst_hbm.at[pl.ds(row0, RW)], dst_v)

        zv = jnp.zeros((16,), jnp.float32)

        def zfill(r, carry):
            for k in range(D // 16):
                zero_v[r, pl.ds(k * 16, 16)] = zv
            return carry

        lax.fori_loop(0, ZR, zfill, 0)
        for t in range(STRIPE // ZR):
            pltpu.sync_copy(zero_v,
                            acc_sh.at[pl.ds(s * STRIPE + t * ZR, ZR)])
        if with_cnt:
            ov = jnp.ones((16,), jnp.float32)
            for o in (0, 16, 32, 48, 64, 80, 96, SUB - 16):
                ones_v[pl.ds(o, 16)] = ov
            for t in range(STRIPE // D):
                pltpu.sync_copy(zero_v.at[0],
                                cnt_sh.at[pl.ds(s * STRIPE + t * D, D)])
        plsc.subcore_barrier()

        def fire(ch, b):
            for j in range(J):
                pltpu.async_copy(p_hbm.at[src_v.at[ch * J + j]],
                                 rows_v.at[b, j], sems[b])

        def drain_scatter(ch, b):
            for j in range(J):
                pltpu.make_async_copy(p_hbm.at[src_v.at[ch * J + j]],
                                      rows_v.at[b, j], sems[b]).wait()
            for j in range(J):
                pltpu.sync_copy(rows_v.at[b, j],
                                acc_sh.at[dst_v.at[ch * J + j]], add=True)
            if with_cnt:
                for j in range(J):
                    pltpu.sync_copy(ones_v, cnt_sh.at[dst_v.at[ch * J + j]],
                                    add=True)

        fire(0, 0)

        def step(i, carry):
            fire(2 * i + 1, 1)
            drain_scatter(2 * i, 0)

            @pl.when(i < cpw // 2 - 1)
            def _():
                fire(2 * i + 2, 0)
            drain_scatter(2 * i + 1, 1)
            return carry

        lax.fori_loop(0, cpw // 2, step, 0)
        plsc.subcore_barrier()

        pltpu.sync_copy(acc_sh.at[pl.ds(s * STRIPE, STRIPE)],
                        part_hbm.at[c, pl.ds(s * STRIPE, STRIPE)])
        if with_cnt:
            @pl.when(s == 0)
            def _():
                pltpu.sync_copy(cnt_sh, cntp_hbm.at[c])

    return pl.kernel(
        body,
        out_type=tuple(out_type),
        mesh=mesh,
        scratch_types=tuple(scratch),
        compiler_params=pltpu.CompilerParams(use_tc_tiling_on_sc=False))


def _proj1_body(x_ref, wl_ref, wr_ref, b_ref, p_ref, r_ref):
    xv = x_ref[...]
    p_ref[...] = jnp.dot(xv, wl_ref[...], preferred_element_type=jnp.float32)
    r_ref[...] = (jnp.dot(xv, wr_ref[...], preferred_element_type=jnp.float32)
                  + b_ref[...])


def _mid_body(part_ref, cnt_ref, r1_ref, wl2_ref, wr2_ref, b2_ref,
              p2_ref, r2_ref, inv_ref):
    a = part_ref[0, :N] + part_ref[1, :N]
    cnt = cnt_ref[0, :N] + cnt_ref[1, :N]
    inv = (1.0 / jnp.maximum(cnt, 1.0))[:, None]
    h = jnp.maximum(a * inv + r1_ref[...], 0.0)
    p2 = jnp.dot(h, wl2_ref[...], preferred_element_type=jnp.float32)
    p2_ref[...] = jnp.concatenate(
        [p2, jnp.zeros((N, 16 - p2.shape[1]), jnp.float32)], axis=1)
    r2_ref[...] = (jnp.dot(h, wr2_ref[...], preferred_element_type=jnp.float32)
                   + b2_ref[...])
    inv_ref[...] = inv


def _out_body(part_ref, inv_ref, r2_ref, o_ref):
    q = part_ref[0, :N, :2] + part_ref[1, :N, :2]
    o_ref[...] = q * inv_ref[...] + r2_ref[...]


def kernel(x, edge_index, W_l1, W_r1, b1, W_l2, W_r2, b2):
    HID = W_l1.shape[1]
    OUT = W_l2.shape[1]
    D2 = 16

    J1, J2 = 4, 10
    src2 = edge_index[0].astype(jnp.int32).reshape(E // SUB, SUB)
    dst2 = edge_index[1].astype(jnp.int32).reshape(E // SUB, SUB)

    p1, r1 = pl.pallas_call(
        _proj1_body,
        out_shape=(jax.ShapeDtypeStruct((N, HID), jnp.float32),
                   jax.ShapeDtypeStruct((N, HID), jnp.float32)),
    )(x, W_l1, W_r1, b1)

    part1, cntp = _seg_sum_kernel(HID, J1, True)(p1, src2, dst2)

    p2, r2, inv = pl.pallas_call(
        _mid_body,
        out_shape=(jax.ShapeDtypeStruct((N, D2), jnp.float32),
                   jax.ShapeDtypeStruct((N, OUT), jnp.float32),
                   jax.ShapeDtypeStruct((N, 1), jnp.float32)),
    )(part1, cntp, r1, W_l2, W_r2, b2)

    (part2,) = _seg_sum_kernel(D2, J2, False)(p2, src2, dst2)

    return pl.pallas_call(
        _out_body,
        out_shape=jax.ShapeDtypeStruct((N, OUT), jnp.float32),
    )(part2, inv, r2)
